# baseline (speedup 1.0000x reference)
import functools

import jax
import jax.numpy as jnp
from jax import lax
from jax.experimental import pallas as pl
from jax.experimental.pallas import tpu as pltpu
from jax.experimental.pallas import tpu_sc as plsc

N = 10000
HIDDEN = 128
NFILT = 128
NINT = 6
NG = 50
CUTOFF = 5.0
MAXNB = 32
E = N * MAXNB

_LN2 = 0.6931471805599453
_GSTEP = CUTOFF / (NG - 1)
_GAMMA = 0.5 / _GSTEP**2

BE = 2000
BN = 200
BH = 2000
BR = 2000


def _ssp(x):
    return jnp.maximum(x, 0.0) + jnp.log1p(jnp.exp(-jnp.abs(x))) - _LN2


_NW = 32
_NPW = 320
_NPAD = _NW * _NPW
_NCH = 625
_CAP = 128
_BIGF = 1e30


def _nbr_body(px_hbm, py_hbm, pz_hbm, col_hbm, d2_hbm,
              xs, ys, zs, bufd, bufi, colst, d2st):
    wid = lax.axis_index("s") * 2 + lax.axis_index("c")
    base = wid * _NPW
    pltpu.sync_copy(px_hbm, xs.at[pl.ds(0, _NPAD)])
    pltpu.sync_copy(py_hbm, ys.at[pl.ds(0, _NPAD)])
    pltpu.sync_copy(pz_hbm, zs.at[pl.ds(0, _NPAD)])
    lanes = lax.broadcasted_iota(jnp.int32, (16,), 0)

    def node_body(i, _):
        n = base + i
        x0 = xs[pl.ds(n, 16)][0]
        y0 = ys[pl.ds(n, 16)][0]
        z0 = zs[pl.ds(n, 16)][0]
        for k in range(_CAP // 16):
            bufd[pl.ds(k * 16, 16)] = jnp.full((16,), _BIGF, jnp.float32)
            bufi[pl.ds(k * 16, 16)] = jnp.full((16,), i, jnp.int32)

        def cand_body(j, cnt):
            c0 = j * 16
            dx = xs[pl.ds(c0, 16)] - x0
            dy = ys[pl.ds(c0, 16)] - y0
            dz = zs[pl.ds(c0, 16)] - z0
            d2 = dx * dx + dy * dy + dz * dz
            ids = c0 + lanes
            m = (d2 <= CUTOFF**2) & (ids != n)
            pos = cnt + plsc.cumsum(m.astype(jnp.int32)) - m.astype(jnp.int32)
            m = m & (pos < _CAP)
            plsc.store_scatter(bufd, [pos], d2, mask=m)
            plsc.store_scatter(bufi, [pos], ids, mask=m)
            return cnt + plsc.all_reduce_population_count(m)

        lax.fori_loop(0, _NCH, cand_body, jnp.zeros((16,), jnp.int32),
                      unroll=4)

        bd = [bufd[pl.ds(k * 16, 16)] for k in range(_CAP // 16)]
        bi = [bufi[pl.ds(k * 16, 16)] for k in range(_CAP // 16)]
        outd = jnp.zeros((16,), jnp.float32)
        outi = jnp.zeros((16,), jnp.int32)
        for s in range(MAXNB):
            mv = bd[0]
            for k in range(1, _CAP // 16):
                mv = jnp.minimum(mv, bd[k])
            minval = jnp.min(mv)
            iv = jnp.where(bd[0] == minval, bi[0], jnp.int32(2**30))
            for k in range(1, _CAP // 16):
                iv = jnp.minimum(iv, jnp.where(bd[k] == minval, bi[k],
                                               jnp.int32(2**30)))
            minidx = jnp.min(iv)
            lane = s % 16
            outd = jnp.where(lanes == lane, minval, outd)
            outi = jnp.where(lanes == lane, minidx, outi)
            if lane == 15:
                d2st[pl.ds(i * MAXNB + (s // 16) * 16, 16)] = outd
                colst[pl.ds(i * MAXNB + (s // 16) * 16, 16)] = outi
            for k in range(_CAP // 16):
                hit = (bd[k] == minval) & (bi[k] == minidx)
                bd[k] = jnp.where(hit, _BIGF, bd[k])
        return 0

    lax.fori_loop(0, _NPW, node_body, 0)
    pltpu.sync_copy(colst, col_hbm.at[pl.ds(base * MAXNB, _NPW * MAXNB)])
    pltpu.sync_copy(d2st, d2_hbm.at[pl.ds(base * MAXNB, _NPW * MAXNB)])


def _nbr_sc(px, py, pz):
    mesh = plsc.VectorSubcoreMesh(core_axis_name="c", subcore_axis_name="s",
                                  num_cores=2, num_subcores=16)
    f = pl.kernel(
        _nbr_body,
        out_type=[
            jax.ShapeDtypeStruct((_NPAD * MAXNB,), jnp.int32),
            jax.ShapeDtypeStruct((_NPAD * MAXNB,), jnp.float32),
        ],
        mesh=mesh,
        compiler_params=pltpu.CompilerParams(needs_layout_passes=False),
        scratch_types=[
            pltpu.VMEM((_NPAD + 16,), jnp.float32),
            pltpu.VMEM((_NPAD + 16,), jnp.float32),
            pltpu.VMEM((_NPAD + 16,), jnp.float32),
            pltpu.VMEM((_CAP,), jnp.float32),
            pltpu.VMEM((_CAP,), jnp.int32),
            pltpu.VMEM((_NPW * MAXNB,), jnp.int32),
            pltpu.VMEM((_NPW * MAXNB,), jnp.float32),
        ],
    )
    return f(px, py, pz)


def _filt_body(dist_ref, cw_ref, w0_ref, b0_ref, w1_ref, b1_ref, out_ref):
    d = dist_ref[...]
    cw = cw_ref[...]
    off = _GSTEP * lax.broadcasted_iota(jnp.int32, (1, NG), 1).astype(jnp.float32)
    attr = jnp.exp(-_GAMMA * (d - off) ** 2)
    for i in range(NINT):
        x = jnp.dot(attr, w0_ref[i], preferred_element_type=jnp.float32)
        x = _ssp(x + b0_ref[i:i + 1, :])
        x = jnp.dot(x, w1_ref[i], preferred_element_type=jnp.float32)
        x = x + b1_ref[i:i + 1, :]
        out_ref[i] = x * cw


def _filters(dist, cw, w0, b0, w1, b1):
    nb = E // BE
    return pl.pallas_call(
        _filt_body,
        grid=(nb,),
        in_specs=[
            pl.BlockSpec((BE, 1), lambda b: (b, 0)),
            pl.BlockSpec((BE, 1), lambda b: (b, 0)),
            pl.BlockSpec((NINT, NG, NFILT), lambda b: (0, 0, 0)),
            pl.BlockSpec((NINT, NFILT), lambda b: (0, 0)),
            pl.BlockSpec((NINT, NFILT, NFILT), lambda b: (0, 0, 0)),
            pl.BlockSpec((NINT, NFILT), lambda b: (0, 0)),
        ],
        out_specs=pl.BlockSpec((NINT, BE, NFILT), lambda b: (0, b, 0)),
        out_shape=jax.ShapeDtypeStruct((NINT, E, NFILT), jnp.float32),
    )(dist, cw, w0, b0, w1, b1)


def _gather_sc(hf, colp, fc_all, layer):
    def _gath_body(hf_hbm, colp_hbm, fc_hbm, agg_hbm,
                   colv, rows, fcb, aggst, sem):
        wid = lax.axis_index("s") * 2 + lax.axis_index("c")
        base = wid * _NPW
        pltpu.sync_copy(colp_hbm.at[pl.ds(base * MAXNB, _NPW * MAXNB)], colv)

        def node_body(i, _):
            n = base + i
            nc = jnp.minimum(n, N - 1)
            cp = pltpu.async_copy(
                hf_hbm.at[colv.at[pl.ds(i * MAXNB, MAXNB)]], rows, sem)
            cp2 = pltpu.async_copy(
                fc_hbm.at[layer, pl.ds(nc * MAXNB, MAXNB), :], fcb, sem)
            cp.wait()
            cp2.wait()
            for c in range(NFILT // 16):
                acc = rows[0, pl.ds(c * 16, 16)] * fcb[0, pl.ds(c * 16, 16)]
                for r in range(1, MAXNB):
                    acc = acc + (rows[r, pl.ds(c * 16, 16)]
                                 * fcb[r, pl.ds(c * 16, 16)])
                aggst[pl.ds(i * NFILT + c * 16, 16)] = acc
            return 0

        lax.fori_loop(0, _NPW, node_body, 0)
        pltpu.sync_copy(aggst, agg_hbm.at[pl.ds(base * NFILT, _NPW * NFILT)])

    mesh = plsc.VectorSubcoreMesh(core_axis_name="c", subcore_axis_name="s",
                                  num_cores=2, num_subcores=16)
    f = pl.kernel(
        _gath_body,
        out_type=jax.ShapeDtypeStruct((_NPAD * NFILT,), jnp.float32),
        mesh=mesh,
        compiler_params=pltpu.CompilerParams(needs_layout_passes=False),
        scratch_types=[
            pltpu.VMEM((_NPW * MAXNB,), jnp.int32),
            pltpu.VMEM((MAXNB, NFILT), jnp.float32),
            pltpu.VMEM((MAXNB, NFILT), jnp.float32),
            pltpu.VMEM((_NPW * NFILT,), jnp.float32),
            pltpu.SemaphoreType.DMA,
        ],
    )
    return f(hf, colp, fc_all).reshape(_NPAD, NFILT)[:N]


def _hf_body(h_ref, w_ref, out_ref):
    out_ref[...] = jnp.dot(h_ref[...], w_ref[0],
                           preferred_element_type=jnp.float32)


def _hf(h, conv_w1, i):
    return pl.pallas_call(
        _hf_body,
        grid=(N // BH,),
        in_specs=[
            pl.BlockSpec((BH, HIDDEN), lambda b: (b, 0)),
            pl.BlockSpec((1, HIDDEN, NFILT), lambda b, _i=i: (_i, 0, 0)),
        ],
        out_specs=pl.BlockSpec((BH, NFILT), lambda b: (b, 0)),
        out_shape=jax.ShapeDtypeStruct((N, NFILT), jnp.float32),
    )(h, conv_w1)


def _upd_body(agg_ref, h_ref, w2_ref, b2_ref, lw_ref, lb_ref, out_ref):
    hc = jnp.dot(agg_ref[...], w2_ref[0], preferred_element_type=jnp.float32)
    hc = _ssp(hc + b2_ref[0:1, 0, :])
    hc = jnp.dot(hc, lw_ref[0], preferred_element_type=jnp.float32)
    out_ref[...] = h_ref[...] + hc + lb_ref[0:1, 0, :]


def _upd(agg, h, conv_w2, conv_b2, lin_w, lin_b, i):
    return pl.pallas_call(
        _upd_body,
        grid=(N // BH,),
        in_specs=[
            pl.BlockSpec((BH, NFILT), lambda b: (b, 0)),
            pl.BlockSpec((BH, HIDDEN), lambda b: (b, 0)),
            pl.BlockSpec((1, NFILT, HIDDEN), lambda b, _i=i: (_i, 0, 0)),
            pl.BlockSpec((1, 1, HIDDEN), lambda b, _i=i: (_i, 0, 0)),
            pl.BlockSpec((1, HIDDEN, HIDDEN), lambda b, _i=i: (_i, 0, 0)),
            pl.BlockSpec((1, 1, HIDDEN), lambda b, _i=i: (_i, 0, 0)),
        ],
        out_specs=pl.BlockSpec((BH, HIDDEN), lambda b: (b, 0)),
        out_shape=jax.ShapeDtypeStruct((N, HIDDEN), jnp.float32),
    )(agg, h, conv_w2, conv_b2, lin_w, lin_b)


def _readout_body(h_ref, w1_ref, b1_ref, w2_ref, b2_ref, out_ref):
    @pl.when(pl.program_id(0) == 0)
    def _():
        out_ref[...] = jnp.zeros_like(out_ref)

    x = _ssp(jnp.dot(h_ref[...], w1_ref[...],
                     preferred_element_type=jnp.float32) + b1_ref[...])
    y = jnp.dot(x, w2_ref[...], preferred_element_type=jnp.float32)
    out_ref[...] += (jnp.sum(y, axis=0, keepdims=True)
                     + BR * b2_ref[...])


def _readout(h, out_w1, out_b1, out_w2, out_b2):
    return pl.pallas_call(
        _readout_body,
        grid=(N // BR,),
        in_specs=[
            pl.BlockSpec((BR, HIDDEN), lambda b: (b, 0)),
            pl.BlockSpec((HIDDEN, HIDDEN // 2), lambda b: (0, 0)),
            pl.BlockSpec((1, HIDDEN // 2), lambda b: (0, 0)),
            pl.BlockSpec((HIDDEN // 2, 1), lambda b: (0, 0)),
            pl.BlockSpec((1, 1), lambda b: (0, 0)),
        ],
        out_specs=pl.BlockSpec((1, 1), lambda b: (0, 0)),
        out_shape=jax.ShapeDtypeStruct((1, 1), jnp.float32),
    )(h, out_w1, out_b1.reshape(1, -1), out_w2, out_b2.reshape(1, 1))


def kernel(z, pos, emb, mlp_w0, mlp_b0, mlp_w1, mlp_b1, conv_w1, conv_w2,
           conv_b2, lin_w, lin_b, out_w1, out_b1, out_w2, out_b2):
    posp = jnp.concatenate(
        [pos, jnp.full((_NPAD - N, 3), 1e9, jnp.float32)], axis=0)
    colp, d2p = _nbr_sc(posp[:, 0], posp[:, 1], posp[:, 2])
    d2s = d2p[:E]
    valid = d2s <= CUTOFF**2
    dist = jnp.sqrt(d2s + 1e-12)
    c = 0.5 * (jnp.cos(dist * (jnp.pi / CUTOFF)) + 1.0)
    cw = c * valid.astype(jnp.float32)

    fc_all = _filters(dist.reshape(E, 1), cw.reshape(E, 1),
                      mlp_w0, mlp_b0, mlp_w1, mlp_b1)

    h = emb[z]
    cb2 = conv_b2.reshape(NINT, 1, HIDDEN)
    lb = lin_b.reshape(NINT, 1, HIDDEN)
    for i in range(NINT):
        hf = _hf(h, conv_w1, i)
        agg = _gather_sc(hf, colp, fc_all, i)
        h = _upd(agg, h, conv_w2, cb2, lin_w, lb, i)

    return _readout(h, out_w1, out_b1, out_w2, out_b2)


# speedup vs baseline: 1.2545x; 1.2545x over previous
import functools

import jax
import jax.numpy as jnp
from jax import lax
from jax.experimental import pallas as pl
from jax.experimental.pallas import tpu as pltpu
from jax.experimental.pallas import tpu_sc as plsc

N = 10000
HIDDEN = 128
NFILT = 128
NINT = 6
NG = 50
CUTOFF = 5.0
MAXNB = 32
E = N * MAXNB

_LN2 = 0.6931471805599453
_GSTEP = CUTOFF / (NG - 1)
_GAMMA = 0.5 / _GSTEP**2

BE = 2000
BN = 200
BH = 2000
BR = 2000


def _ssp(x):
    return jnp.maximum(x, 0.0) + jnp.log1p(jnp.exp(-jnp.abs(x))) - _LN2


_NW = 32
_NPW = 320
_NPAD = _NW * _NPW
_NCH = 625
_CAP = 128
_BIGF = 1e30


def _nbr_body(px_hbm, py_hbm, pz_hbm, col_hbm, d2_hbm,
              xs, ys, zs, bufd, bufi, colst, d2st):
    wid = lax.axis_index("s") * 2 + lax.axis_index("c")
    base = wid * _NPW
    pltpu.sync_copy(px_hbm, xs.at[pl.ds(0, _NPAD)])
    pltpu.sync_copy(py_hbm, ys.at[pl.ds(0, _NPAD)])
    pltpu.sync_copy(pz_hbm, zs.at[pl.ds(0, _NPAD)])
    lanes = lax.broadcasted_iota(jnp.int32, (16,), 0)

    def node_body(i, _):
        n = base + i
        x0 = xs[pl.ds(n, 16)][0]
        y0 = ys[pl.ds(n, 16)][0]
        z0 = zs[pl.ds(n, 16)][0]
        for k in range(_CAP // 16):
            bufd[pl.ds(k * 16, 16)] = jnp.full((16,), _BIGF, jnp.float32)
            bufi[pl.ds(k * 16, 16)] = jnp.full((16,), i, jnp.int32)

        def cand_body(j, cnt):
            c0 = j * 16
            dx = xs[pl.ds(c0, 16)] - x0
            dy = ys[pl.ds(c0, 16)] - y0
            dz = zs[pl.ds(c0, 16)] - z0
            d2 = dx * dx + dy * dy + dz * dz
            ids = c0 + lanes
            m = (d2 <= CUTOFF**2) & (ids != n)
            pos = cnt + plsc.cumsum(m.astype(jnp.int32)) - m.astype(jnp.int32)
            m = m & (pos < _CAP)
            plsc.store_scatter(bufd, [pos], d2, mask=m)
            plsc.store_scatter(bufi, [pos], ids, mask=m)
            return cnt + plsc.all_reduce_population_count(m)

        lax.fori_loop(0, _NCH, cand_body, jnp.zeros((16,), jnp.int32),
                      unroll=4)

        bd = [bufd[pl.ds(k * 16, 16)] for k in range(_CAP // 16)]
        bi = [bufi[pl.ds(k * 16, 16)] for k in range(_CAP // 16)]
        outd = jnp.zeros((16,), jnp.float32)
        outi = jnp.zeros((16,), jnp.int32)
        for s in range(MAXNB):
            mv = bd[0]
            for k in range(1, _CAP // 16):
                mv = jnp.minimum(mv, bd[k])
            minval = jnp.min(mv)
            iv = jnp.where(bd[0] == minval, bi[0], jnp.int32(2**30))
            for k in range(1, _CAP // 16):
                iv = jnp.minimum(iv, jnp.where(bd[k] == minval, bi[k],
                                               jnp.int32(2**30)))
            minidx = jnp.min(iv)
            lane = s % 16
            outd = jnp.where(lanes == lane, minval, outd)
            outi = jnp.where(lanes == lane, minidx, outi)
            if lane == 15:
                d2st[pl.ds(i * MAXNB + (s // 16) * 16, 16)] = outd
                colst[pl.ds(i * MAXNB + (s // 16) * 16, 16)] = outi
            for k in range(_CAP // 16):
                hit = (bd[k] == minval) & (bi[k] == minidx)
                bd[k] = jnp.where(hit, _BIGF, bd[k])
        return 0

    lax.fori_loop(0, _NPW, node_body, 0)
    pltpu.sync_copy(colst, col_hbm.at[pl.ds(base * MAXNB, _NPW * MAXNB)])
    pltpu.sync_copy(d2st, d2_hbm.at[pl.ds(base * MAXNB, _NPW * MAXNB)])


def _nbr_sc(px, py, pz):
    mesh = plsc.VectorSubcoreMesh(core_axis_name="c", subcore_axis_name="s",
                                  num_cores=2, num_subcores=16)
    f = pl.kernel(
        _nbr_body,
        out_type=[
            jax.ShapeDtypeStruct((_NPAD * MAXNB,), jnp.int32),
            jax.ShapeDtypeStruct((_NPAD * MAXNB,), jnp.float32),
        ],
        mesh=mesh,
        compiler_params=pltpu.CompilerParams(needs_layout_passes=False),
        scratch_types=[
            pltpu.VMEM((_NPAD + 16,), jnp.float32),
            pltpu.VMEM((_NPAD + 16,), jnp.float32),
            pltpu.VMEM((_NPAD + 16,), jnp.float32),
            pltpu.VMEM((_CAP,), jnp.float32),
            pltpu.VMEM((_CAP,), jnp.int32),
            pltpu.VMEM((_NPW * MAXNB,), jnp.int32),
            pltpu.VMEM((_NPW * MAXNB,), jnp.float32),
        ],
    )
    return f(px, py, pz)


def _filt_body(dist_ref, cw_ref, w0_ref, b0_ref, w1_ref, b1_ref, out_ref):
    d = dist_ref[...]
    cw = cw_ref[...]
    off = _GSTEP * lax.broadcasted_iota(jnp.int32, (1, NG), 1).astype(jnp.float32)
    attr = jnp.exp(-_GAMMA * (d - off) ** 2)
    for i in range(NINT):
        x = jnp.dot(attr, w0_ref[i], preferred_element_type=jnp.float32)
        x = _ssp(x + b0_ref[i:i + 1, :])
        x = jnp.dot(x, w1_ref[i], preferred_element_type=jnp.float32)
        x = x + b1_ref[i:i + 1, :]
        out_ref[i] = x * cw


def _filters(dist, cw, w0, b0, w1, b1):
    nb = E // BE
    return pl.pallas_call(
        _filt_body,
        grid=(nb,),
        in_specs=[
            pl.BlockSpec((BE, 1), lambda b: (b, 0)),
            pl.BlockSpec((BE, 1), lambda b: (b, 0)),
            pl.BlockSpec((NINT, NG, NFILT), lambda b: (0, 0, 0)),
            pl.BlockSpec((NINT, NFILT), lambda b: (0, 0)),
            pl.BlockSpec((NINT, NFILT, NFILT), lambda b: (0, 0, 0)),
            pl.BlockSpec((NINT, NFILT), lambda b: (0, 0)),
        ],
        out_specs=pl.BlockSpec((NINT, BE, NFILT), lambda b: (0, b, 0)),
        out_shape=jax.ShapeDtypeStruct((NINT, E, NFILT), jnp.float32),
    )(dist, cw, w0, b0, w1, b1)


def _gather_sc(hf, colp, fc_all, layer):
    def _gath_body(hf_hbm, colp_hbm, fc_hbm, agg_hbm,
                   colv, rows0, fcb0, rows1, fcb1, aggst, sem0, sem1):
        wid = lax.axis_index("s") * 2 + lax.axis_index("c")
        base = wid * _NPW
        pltpu.sync_copy(colp_hbm.at[pl.ds(base * MAXNB, _NPW * MAXNB)], colv)
        bufs = ((rows0, fcb0, sem0), (rows1, fcb1, sem1))

        def issue(i, b):
            rows_b, fcb_b, sem_b = bufs[b]
            ic = jnp.minimum(i, _NPW - 1)
            nc = jnp.minimum(base + ic, N - 1)
            pltpu.async_copy(
                hf_hbm.at[colv.at[pl.ds(ic * MAXNB, MAXNB)]], rows_b, sem_b)
            pltpu.async_copy(
                fc_hbm.at[layer, pl.ds(nc * MAXNB, MAXNB), :], fcb_b, sem_b)

        def waitpair(b):
            rows_b, fcb_b, sem_b = bufs[b]
            pltpu.make_async_copy(
                hf_hbm.at[pl.ds(0, MAXNB), :], rows_b, sem_b).wait()
            pltpu.make_async_copy(
                fc_hbm.at[layer, pl.ds(0, MAXNB), :], fcb_b, sem_b).wait()

        def compute(i, b):
            rows_b, fcb_b, _ = bufs[b]
            for c in range(NFILT // 16):
                acc = rows_b[0, pl.ds(c * 16, 16)] * fcb_b[0, pl.ds(c * 16, 16)]
                for r in range(1, MAXNB):
                    acc = acc + (rows_b[r, pl.ds(c * 16, 16)]
                                 * fcb_b[r, pl.ds(c * 16, 16)])
                aggst[pl.ds(i * NFILT + c * 16, 16)] = acc

        issue(0, 0)

        def pair_body(p, _):
            i0 = p * 2
            issue(i0 + 1, 1)
            waitpair(0)
            compute(i0, 0)
            issue(i0 + 2, 0)
            waitpair(1)
            compute(i0 + 1, 1)
            return 0

        lax.fori_loop(0, _NPW // 2, pair_body, 0)
        waitpair(0)
        pltpu.sync_copy(aggst, agg_hbm.at[pl.ds(base * NFILT, _NPW * NFILT)])

    mesh = plsc.VectorSubcoreMesh(core_axis_name="c", subcore_axis_name="s",
                                  num_cores=2, num_subcores=16)
    f = pl.kernel(
        _gath_body,
        out_type=jax.ShapeDtypeStruct((_NPAD * NFILT,), jnp.float32),
        mesh=mesh,
        compiler_params=pltpu.CompilerParams(needs_layout_passes=False),
        scratch_types=[
            pltpu.VMEM((_NPW * MAXNB,), jnp.int32),
            pltpu.VMEM((MAXNB, NFILT), jnp.float32),
            pltpu.VMEM((MAXNB, NFILT), jnp.float32),
            pltpu.VMEM((MAXNB, NFILT), jnp.float32),
            pltpu.VMEM((MAXNB, NFILT), jnp.float32),
            pltpu.VMEM((_NPW * NFILT,), jnp.float32),
            pltpu.SemaphoreType.DMA,
            pltpu.SemaphoreType.DMA,
        ],
    )
    return f(hf, colp, fc_all).reshape(_NPAD, NFILT)[:N]


def _hf_body(h_ref, w_ref, out_ref):
    out_ref[...] = jnp.dot(h_ref[...], w_ref[0],
                           preferred_element_type=jnp.float32)


def _hf(h, conv_w1, i):
    return pl.pallas_call(
        _hf_body,
        grid=(N // BH,),
        in_specs=[
            pl.BlockSpec((BH, HIDDEN), lambda b: (b, 0)),
            pl.BlockSpec((1, HIDDEN, NFILT), lambda b, _i=i: (_i, 0, 0)),
        ],
        out_specs=pl.BlockSpec((BH, NFILT), lambda b: (b, 0)),
        out_shape=jax.ShapeDtypeStruct((N, NFILT), jnp.float32),
    )(h, conv_w1)


def _upd_body(agg_ref, h_ref, w2_ref, b2_ref, lw_ref, lb_ref, out_ref):
    hc = jnp.dot(agg_ref[...], w2_ref[0], preferred_element_type=jnp.float32)
    hc = _ssp(hc + b2_ref[0:1, 0, :])
    hc = jnp.dot(hc, lw_ref[0], preferred_element_type=jnp.float32)
    out_ref[...] = h_ref[...] + hc + lb_ref[0:1, 0, :]


def _upd(agg, h, conv_w2, conv_b2, lin_w, lin_b, i):
    return pl.pallas_call(
        _upd_body,
        grid=(N // BH,),
        in_specs=[
            pl.BlockSpec((BH, NFILT), lambda b: (b, 0)),
            pl.BlockSpec((BH, HIDDEN), lambda b: (b, 0)),
            pl.BlockSpec((1, NFILT, HIDDEN), lambda b, _i=i: (_i, 0, 0)),
            pl.BlockSpec((1, 1, HIDDEN), lambda b, _i=i: (_i, 0, 0)),
            pl.BlockSpec((1, HIDDEN, HIDDEN), lambda b, _i=i: (_i, 0, 0)),
            pl.BlockSpec((1, 1, HIDDEN), lambda b, _i=i: (_i, 0, 0)),
        ],
        out_specs=pl.BlockSpec((BH, HIDDEN), lambda b: (b, 0)),
        out_shape=jax.ShapeDtypeStruct((N, HIDDEN), jnp.float32),
    )(agg, h, conv_w2, conv_b2, lin_w, lin_b)


def _readout_body(h_ref, w1_ref, b1_ref, w2_ref, b2_ref, out_ref):
    @pl.when(pl.program_id(0) == 0)
    def _():
        out_ref[...] = jnp.zeros_like(out_ref)

    x = _ssp(jnp.dot(h_ref[...], w1_ref[...],
                     preferred_element_type=jnp.float32) + b1_ref[...])
    y = jnp.dot(x, w2_ref[...], preferred_element_type=jnp.float32)
    out_ref[...] += (jnp.sum(y, axis=0, keepdims=True)
                     + BR * b2_ref[...])


def _readout(h, out_w1, out_b1, out_w2, out_b2):
    return pl.pallas_call(
        _readout_body,
        grid=(N // BR,),
        in_specs=[
            pl.BlockSpec((BR, HIDDEN), lambda b: (b, 0)),
            pl.BlockSpec((HIDDEN, HIDDEN // 2), lambda b: (0, 0)),
            pl.BlockSpec((1, HIDDEN // 2), lambda b: (0, 0)),
            pl.BlockSpec((HIDDEN // 2, 1), lambda b: (0, 0)),
            pl.BlockSpec((1, 1), lambda b: (0, 0)),
        ],
        out_specs=pl.BlockSpec((1, 1), lambda b: (0, 0)),
        out_shape=jax.ShapeDtypeStruct((1, 1), jnp.float32),
    )(h, out_w1, out_b1.reshape(1, -1), out_w2, out_b2.reshape(1, 1))


def kernel(z, pos, emb, mlp_w0, mlp_b0, mlp_w1, mlp_b1, conv_w1, conv_w2,
           conv_b2, lin_w, lin_b, out_w1, out_b1, out_w2, out_b2):
    posp = jnp.concatenate(
        [pos, jnp.full((_NPAD - N, 3), 1e9, jnp.float32)], axis=0)
    colp, d2p = _nbr_sc(posp[:, 0], posp[:, 1], posp[:, 2])
    d2s = d2p[:E]
    valid = d2s <= CUTOFF**2
    dist = jnp.sqrt(d2s + 1e-12)
    c = 0.5 * (jnp.cos(dist * (jnp.pi / CUTOFF)) + 1.0)
    cw = c * valid.astype(jnp.float32)

    fc_all = _filters(dist.reshape(E, 1), cw.reshape(E, 1),
                      mlp_w0, mlp_b0, mlp_w1, mlp_b1)

    h = emb[z]
    cb2 = conv_b2.reshape(NINT, 1, HIDDEN)
    lb = lin_b.reshape(NINT, 1, HIDDEN)
    for i in range(NINT):
        hf = _hf(h, conv_w1, i)
        agg = _gather_sc(hf, colp, fc_all, i)
        h = _upd(agg, h, conv_w2, cb2, lin_w, lb, i)

    return _readout(h, out_w1, out_b1, out_w2, out_b2)


# speedup vs baseline: 1.4487x; 1.1548x over previous
import functools

import jax
import jax.numpy as jnp
from jax import lax
from jax.experimental import pallas as pl
from jax.experimental.pallas import tpu as pltpu
from jax.experimental.pallas import tpu_sc as plsc

N = 10000
HIDDEN = 128
NFILT = 128
NINT = 6
NG = 50
CUTOFF = 5.0
MAXNB = 32
E = N * MAXNB

_LN2 = 0.6931471805599453
_GSTEP = CUTOFF / (NG - 1)
_GAMMA = 0.5 / _GSTEP**2

BE = 2000
BN = 200
BH = 2000
BR = 2000


def _ssp(x):
    return jnp.maximum(x, 0.0) + jnp.log1p(jnp.exp(-jnp.abs(x))) - _LN2


_NW = 32
_NPW = 320
_NPAD = _NW * _NPW
_NCH = 625
_CAP = 128
_BIGF = 1e30


def _nbr_body(px_hbm, py_hbm, pz_hbm, jc0_hbm, jc1_hbm, col_hbm, d2_hbm,
              xs, ys, zs, jc0v, jc1v, bufd, bufi, colst, d2st):
    wid = lax.axis_index("s") * 2 + lax.axis_index("c")
    base = wid * _NPW
    pltpu.sync_copy(px_hbm, xs.at[pl.ds(0, _NPAD)])
    pltpu.sync_copy(py_hbm, ys.at[pl.ds(0, _NPAD)])
    pltpu.sync_copy(pz_hbm, zs.at[pl.ds(0, _NPAD)])
    pltpu.sync_copy(jc0_hbm.at[pl.ds(base, _NPW)], jc0v.at[pl.ds(0, _NPW)])
    pltpu.sync_copy(jc1_hbm.at[pl.ds(base, _NPW)], jc1v.at[pl.ds(0, _NPW)])
    lanes = lax.broadcasted_iota(jnp.int32, (16,), 0)

    def node_body(i, _):
        n = base + i
        x0 = xs[pl.ds(n, 16)][0]
        y0 = ys[pl.ds(n, 16)][0]
        z0 = zs[pl.ds(n, 16)][0]
        jc0 = jc0v[pl.ds(i, 16)][0]
        jc1 = jc1v[pl.ds(i, 16)][0]
        for k in range(_CAP // 16):
            bufd[pl.ds(k * 16, 16)] = jnp.full((16,), _BIGF, jnp.float32)
            bufi[pl.ds(k * 16, 16)] = jnp.full((16,), i, jnp.int32)

        def cand_body(j, cnt):
            c0 = j * 16
            dx = xs[pl.ds(c0, 16)] - x0
            dy = ys[pl.ds(c0, 16)] - y0
            dz = zs[pl.ds(c0, 16)] - z0
            d2 = dx * dx + dy * dy + dz * dz
            ids = c0 + lanes
            m = (d2 <= CUTOFF**2) & (ids != n)
            pos = cnt + plsc.cumsum(m.astype(jnp.int32)) - m.astype(jnp.int32)
            m = m & (pos < _CAP)
            plsc.store_scatter(bufd, [pos], d2, mask=m)
            plsc.store_scatter(bufi, [pos], ids, mask=m)
            return cnt + plsc.all_reduce_population_count(m)

        lax.fori_loop(jc0, jc1, cand_body, jnp.zeros((16,), jnp.int32))

        bd = [bufd[pl.ds(k * 16, 16)] for k in range(_CAP // 16)]
        bi = [bufi[pl.ds(k * 16, 16)] for k in range(_CAP // 16)]
        outd = jnp.zeros((16,), jnp.float32)
        outi = jnp.zeros((16,), jnp.int32)
        for s in range(MAXNB):
            mv = bd[0]
            for k in range(1, _CAP // 16):
                mv = jnp.minimum(mv, bd[k])
            minval = jnp.min(mv)
            iv = jnp.where(bd[0] == minval, bi[0], jnp.int32(2**30))
            for k in range(1, _CAP // 16):
                iv = jnp.minimum(iv, jnp.where(bd[k] == minval, bi[k],
                                               jnp.int32(2**30)))
            minidx = jnp.min(iv)
            lane = s % 16
            outd = jnp.where(lanes == lane, minval, outd)
            outi = jnp.where(lanes == lane, minidx, outi)
            if lane == 15:
                d2st[pl.ds(i * MAXNB + (s // 16) * 16, 16)] = outd
                colst[pl.ds(i * MAXNB + (s // 16) * 16, 16)] = outi
            for k in range(_CAP // 16):
                hit = (bd[k] == minval) & (bi[k] == minidx)
                bd[k] = jnp.where(hit, _BIGF, bd[k])
        return 0

    lax.fori_loop(0, _NPW, node_body, 0)
    pltpu.sync_copy(colst, col_hbm.at[pl.ds(base * MAXNB, _NPW * MAXNB)])
    pltpu.sync_copy(d2st, d2_hbm.at[pl.ds(base * MAXNB, _NPW * MAXNB)])


def _nbr_sc(px, py, pz, jc0, jc1):
    mesh = plsc.VectorSubcoreMesh(core_axis_name="c", subcore_axis_name="s",
                                  num_cores=2, num_subcores=16)
    f = pl.kernel(
        _nbr_body,
        out_type=[
            jax.ShapeDtypeStruct((_NPAD * MAXNB,), jnp.int32),
            jax.ShapeDtypeStruct((_NPAD * MAXNB,), jnp.float32),
        ],
        mesh=mesh,
        compiler_params=pltpu.CompilerParams(needs_layout_passes=False),
        scratch_types=[
            pltpu.VMEM((_NPAD + 16,), jnp.float32),
            pltpu.VMEM((_NPAD + 16,), jnp.float32),
            pltpu.VMEM((_NPAD + 16,), jnp.float32),
            pltpu.VMEM((_NPW + 16,), jnp.int32),
            pltpu.VMEM((_NPW + 16,), jnp.int32),
            pltpu.VMEM((_CAP,), jnp.float32),
            pltpu.VMEM((_CAP,), jnp.int32),
            pltpu.VMEM((_NPW * MAXNB,), jnp.int32),
            pltpu.VMEM((_NPW * MAXNB,), jnp.float32),
        ],
    )
    return f(px, py, pz, jc0, jc1)


def _filt_body(dist_ref, cw_ref, w0_ref, b0_ref, w1_ref, b1_ref, out_ref):
    d = dist_ref[...]
    cw = cw_ref[...]
    off = _GSTEP * lax.broadcasted_iota(jnp.int32, (1, NG), 1).astype(jnp.float32)
    attr = jnp.exp(-_GAMMA * (d - off) ** 2)
    for i in range(NINT):
        x = jnp.dot(attr, w0_ref[i], preferred_element_type=jnp.float32)
        x = _ssp(x + b0_ref[i:i + 1, :])
        x = jnp.dot(x, w1_ref[i], preferred_element_type=jnp.float32)
        x = x + b1_ref[i:i + 1, :]
        out_ref[i] = x * cw


def _filters(dist, cw, w0, b0, w1, b1):
    nb = E // BE
    return pl.pallas_call(
        _filt_body,
        grid=(nb,),
        in_specs=[
            pl.BlockSpec((BE, 1), lambda b: (b, 0)),
            pl.BlockSpec((BE, 1), lambda b: (b, 0)),
            pl.BlockSpec((NINT, NG, NFILT), lambda b: (0, 0, 0)),
            pl.BlockSpec((NINT, NFILT), lambda b: (0, 0)),
            pl.BlockSpec((NINT, NFILT, NFILT), lambda b: (0, 0, 0)),
            pl.BlockSpec((NINT, NFILT), lambda b: (0, 0)),
        ],
        out_specs=pl.BlockSpec((NINT, BE, NFILT), lambda b: (0, b, 0)),
        out_shape=jax.ShapeDtypeStruct((NINT, E, NFILT), jnp.float32),
    )(dist, cw, w0, b0, w1, b1)


def _gather_sc(hf, colp, fc_all, layer):
    def _gath_body(hf_hbm, colp_hbm, fc_hbm, agg_hbm,
                   colv, rows0, fcb0, rows1, fcb1, aggst, sem0, sem1):
        wid = lax.axis_index("s") * 2 + lax.axis_index("c")
        base = wid * _NPW
        pltpu.sync_copy(colp_hbm.at[pl.ds(base * MAXNB, _NPW * MAXNB)], colv)
        bufs = ((rows0, fcb0, sem0), (rows1, fcb1, sem1))

        def issue(i, b):
            rows_b, fcb_b, sem_b = bufs[b]
            ic = jnp.minimum(i, _NPW - 1)
            nc = jnp.minimum(base + ic, N - 1)
            pltpu.async_copy(
                hf_hbm.at[colv.at[pl.ds(ic * MAXNB, MAXNB)]], rows_b, sem_b)
            pltpu.async_copy(
                fc_hbm.at[layer, pl.ds(nc * MAXNB, MAXNB), :], fcb_b, sem_b)

        def waitpair(b):
            rows_b, fcb_b, sem_b = bufs[b]
            pltpu.make_async_copy(
                hf_hbm.at[pl.ds(0, MAXNB), :], rows_b, sem_b).wait()
            pltpu.make_async_copy(
                fc_hbm.at[layer, pl.ds(0, MAXNB), :], fcb_b, sem_b).wait()

        def compute(i, b):
            rows_b, fcb_b, _ = bufs[b]
            for c in range(NFILT // 16):
                acc = rows_b[0, pl.ds(c * 16, 16)] * fcb_b[0, pl.ds(c * 16, 16)]
                for r in range(1, MAXNB):
                    acc = acc + (rows_b[r, pl.ds(c * 16, 16)]
                                 * fcb_b[r, pl.ds(c * 16, 16)])
                aggst[pl.ds(i * NFILT + c * 16, 16)] = acc

        issue(0, 0)

        def pair_body(p, _):
            i0 = p * 2
            issue(i0 + 1, 1)
            waitpair(0)
            compute(i0, 0)
            issue(i0 + 2, 0)
            waitpair(1)
            compute(i0 + 1, 1)
            return 0

        lax.fori_loop(0, _NPW // 2, pair_body, 0)
        waitpair(0)
        pltpu.sync_copy(aggst, agg_hbm.at[pl.ds(base * NFILT, _NPW * NFILT)])

    mesh = plsc.VectorSubcoreMesh(core_axis_name="c", subcore_axis_name="s",
                                  num_cores=2, num_subcores=16)
    f = pl.kernel(
        _gath_body,
        out_type=jax.ShapeDtypeStruct((_NPAD * NFILT,), jnp.float32),
        mesh=mesh,
        compiler_params=pltpu.CompilerParams(needs_layout_passes=False),
        scratch_types=[
            pltpu.VMEM((_NPW * MAXNB,), jnp.int32),
            pltpu.VMEM((MAXNB, NFILT), jnp.float32),
            pltpu.VMEM((MAXNB, NFILT), jnp.float32),
            pltpu.VMEM((MAXNB, NFILT), jnp.float32),
            pltpu.VMEM((MAXNB, NFILT), jnp.float32),
            pltpu.VMEM((_NPW * NFILT,), jnp.float32),
            pltpu.SemaphoreType.DMA,
            pltpu.SemaphoreType.DMA,
        ],
    )
    return f(hf, colp, fc_all).reshape(_NPAD, NFILT)[:N]


def _hf_body(h_ref, w_ref, out_ref):
    out_ref[...] = jnp.dot(h_ref[...], w_ref[0],
                           preferred_element_type=jnp.float32)


def _hf(h, conv_w1, i):
    return pl.pallas_call(
        _hf_body,
        grid=(N // BH,),
        in_specs=[
            pl.BlockSpec((BH, HIDDEN), lambda b: (b, 0)),
            pl.BlockSpec((1, HIDDEN, NFILT), lambda b, _i=i: (_i, 0, 0)),
        ],
        out_specs=pl.BlockSpec((BH, NFILT), lambda b: (b, 0)),
        out_shape=jax.ShapeDtypeStruct((N, NFILT), jnp.float32),
    )(h, conv_w1)


def _upd_body(agg_ref, h_ref, w2_ref, b2_ref, lw_ref, lb_ref, out_ref):
    hc = jnp.dot(agg_ref[...], w2_ref[0], preferred_element_type=jnp.float32)
    hc = _ssp(hc + b2_ref[0:1, 0, :])
    hc = jnp.dot(hc, lw_ref[0], preferred_element_type=jnp.float32)
    out_ref[...] = h_ref[...] + hc + lb_ref[0:1, 0, :]


def _upd(agg, h, conv_w2, conv_b2, lin_w, lin_b, i):
    return pl.pallas_call(
        _upd_body,
        grid=(N // BH,),
        in_specs=[
            pl.BlockSpec((BH, NFILT), lambda b: (b, 0)),
            pl.BlockSpec((BH, HIDDEN), lambda b: (b, 0)),
            pl.BlockSpec((1, NFILT, HIDDEN), lambda b, _i=i: (_i, 0, 0)),
            pl.BlockSpec((1, 1, HIDDEN), lambda b, _i=i: (_i, 0, 0)),
            pl.BlockSpec((1, HIDDEN, HIDDEN), lambda b, _i=i: (_i, 0, 0)),
            pl.BlockSpec((1, 1, HIDDEN), lambda b, _i=i: (_i, 0, 0)),
        ],
        out_specs=pl.BlockSpec((BH, HIDDEN), lambda b: (b, 0)),
        out_shape=jax.ShapeDtypeStruct((N, HIDDEN), jnp.float32),
    )(agg, h, conv_w2, conv_b2, lin_w, lin_b)


def _readout_body(h_ref, w1_ref, b1_ref, w2_ref, b2_ref, out_ref):
    @pl.when(pl.program_id(0) == 0)
    def _():
        out_ref[...] = jnp.zeros_like(out_ref)

    x = _ssp(jnp.dot(h_ref[...], w1_ref[...],
                     preferred_element_type=jnp.float32) + b1_ref[...])
    y = jnp.dot(x, w2_ref[...], preferred_element_type=jnp.float32)
    out_ref[...] += (jnp.sum(y, axis=0, keepdims=True)
                     + BR * b2_ref[...])


def _readout(h, out_w1, out_b1, out_w2, out_b2):
    return pl.pallas_call(
        _readout_body,
        grid=(N // BR,),
        in_specs=[
            pl.BlockSpec((BR, HIDDEN), lambda b: (b, 0)),
            pl.BlockSpec((HIDDEN, HIDDEN // 2), lambda b: (0, 0)),
            pl.BlockSpec((1, HIDDEN // 2), lambda b: (0, 0)),
            pl.BlockSpec((HIDDEN // 2, 1), lambda b: (0, 0)),
            pl.BlockSpec((1, 1), lambda b: (0, 0)),
        ],
        out_specs=pl.BlockSpec((1, 1), lambda b: (0, 0)),
        out_shape=jax.ShapeDtypeStruct((1, 1), jnp.float32),
    )(h, out_w1, out_b1.reshape(1, -1), out_w2, out_b2.reshape(1, 1))


def kernel(z, pos, emb, mlp_w0, mlp_b0, mlp_w1, mlp_b1, conv_w1, conv_w2,
           conv_b2, lin_w, lin_b, out_w1, out_b1, out_w2, out_b2):
    order = jnp.argsort(pos[:, 0]).astype(jnp.int32)
    pos_s = pos[order]
    xs_s = pos_s[:, 0]
    posp = jnp.concatenate(
        [pos_s, jnp.full((_NPAD - N, 3), 1e9, jnp.float32)], axis=0)
    start = jnp.searchsorted(xs_s, xs_s - CUTOFF, side="left").astype(jnp.int32)
    end = jnp.searchsorted(xs_s, xs_s + CUTOFF, side="right").astype(jnp.int32)
    zpad = jnp.zeros((_NPAD - N,), jnp.int32)
    jc0 = jnp.concatenate([start // 16, zpad])
    jc1 = jnp.concatenate([(end + 15) // 16, zpad])
    colp_s, d2p_s = _nbr_sc(posp[:, 0], posp[:, 1], posp[:, 2], jc0, jc1)

    inv = jnp.zeros((N,), jnp.int32).at[order].set(
        jnp.arange(N, dtype=jnp.int32))
    col_rows = order[colp_s.reshape(_NPAD, MAXNB)[:N]][inv]
    d2s = d2p_s.reshape(_NPAD, MAXNB)[:N][inv].reshape(E)
    padrows = (jnp.arange((_NPAD - N) * MAXNB, dtype=jnp.int32) % N
               ).reshape(_NPAD - N, MAXNB)
    colp = jnp.concatenate([col_rows, padrows], axis=0).reshape(-1)

    valid = d2s <= CUTOFF**2
    dist = jnp.sqrt(d2s + 1e-12)
    c = 0.5 * (jnp.cos(dist * (jnp.pi / CUTOFF)) + 1.0)
    cw = c * valid.astype(jnp.float32)

    fc_all = _filters(dist.reshape(E, 1), cw.reshape(E, 1),
                      mlp_w0, mlp_b0, mlp_w1, mlp_b1)

    h = emb[z]
    cb2 = conv_b2.reshape(NINT, 1, HIDDEN)
    lb = lin_b.reshape(NINT, 1, HIDDEN)
    for i in range(NINT):
        hf = _hf(h, conv_w1, i)
        agg = _gather_sc(hf, colp, fc_all, i)
        h = _upd(agg, h, conv_w2, cb2, lin_w, lb, i)

    return _readout(h, out_w1, out_b1, out_w2, out_b2)


# speedup vs baseline: 2.1843x; 1.5077x over previous
import functools

import jax
import jax.numpy as jnp
from jax import lax
from jax.experimental import pallas as pl
from jax.experimental.pallas import tpu as pltpu
from jax.experimental.pallas import tpu_sc as plsc

N = 10000
HIDDEN = 128
NFILT = 128
NINT = 6
NG = 50
CUTOFF = 5.0
MAXNB = 32
E = N * MAXNB

_LN2 = 0.6931471805599453
_GSTEP = CUTOFF / (NG - 1)
_GAMMA = 0.5 / _GSTEP**2

BE = 2000
BN = 200
BH = 2000
BR = 2000


def _ssp(x):
    return jnp.maximum(x, 0.0) + jnp.log1p(jnp.exp(-jnp.abs(x))) - _LN2


_NW = 32
_NPW = 320
_NPAD = _NW * _NPW
_NCH = 625
_CAP = 128
_BIGF = 1e30


def _nbr_body(px_hbm, py_hbm, pz_hbm, col_hbm, d2_hbm,
              xs, ys, zs, bufd, bufi, colst, d2st):
    wid = lax.axis_index("s") * 2 + lax.axis_index("c")
    base = wid * _NPW
    pltpu.sync_copy(px_hbm, xs.at[pl.ds(0, _NPAD)])
    pltpu.sync_copy(py_hbm, ys.at[pl.ds(0, _NPAD)])
    pltpu.sync_copy(pz_hbm, zs.at[pl.ds(0, _NPAD)])
    lanes = lax.broadcasted_iota(jnp.int32, (16,), 0)

    def _bsearch(target, right):
        def it(_, lh):
            lo, hi = lh
            mid = (lo + hi) // 2
            v = xs[pl.ds(mid, 16)][0]
            go = (v <= target) if right else (v < target)
            return (jnp.where(go, mid + 1, lo), jnp.where(go, hi, mid))

        lo, _hi = lax.fori_loop(0, 14, it, (jnp.int32(0), jnp.int32(N)))
        return lo

    def node_body(i, _):
        n = base + i
        x0 = xs[pl.ds(n, 16)][0]
        y0 = ys[pl.ds(n, 16)][0]
        z0 = zs[pl.ds(n, 16)][0]
        jc0 = _bsearch(x0 - CUTOFF, False) // 16
        jc1 = (_bsearch(x0 + CUTOFF, True) + 15) // 16
        for k in range(_CAP // 16):
            bufd[pl.ds(k * 16, 16)] = jnp.full((16,), _BIGF, jnp.float32)
            bufi[pl.ds(k * 16, 16)] = jnp.full((16,), i, jnp.int32)

        def cand_body(j, cnt):
            c0 = j * 16
            dx = xs[pl.ds(c0, 16)] - x0
            dy = ys[pl.ds(c0, 16)] - y0
            dz = zs[pl.ds(c0, 16)] - z0
            d2 = dx * dx + dy * dy + dz * dz
            ids = c0 + lanes
            m = (d2 <= CUTOFF**2) & (ids != n)
            pos = cnt + plsc.cumsum(m.astype(jnp.int32)) - m.astype(jnp.int32)
            m = m & (pos < _CAP)
            plsc.store_scatter(bufd, [pos], d2, mask=m)
            plsc.store_scatter(bufi, [pos], ids, mask=m)
            return cnt + plsc.all_reduce_population_count(m)

        lax.fori_loop(jc0, jc1, cand_body, jnp.zeros((16,), jnp.int32))

        bd = [bufd[pl.ds(k * 16, 16)] for k in range(_CAP // 16)]
        bi = [bufi[pl.ds(k * 16, 16)] for k in range(_CAP // 16)]
        outd = jnp.zeros((16,), jnp.float32)
        outi = jnp.zeros((16,), jnp.int32)
        for s in range(MAXNB):
            mv = bd[0]
            for k in range(1, _CAP // 16):
                mv = jnp.minimum(mv, bd[k])
            minval = jnp.min(mv)
            iv = jnp.where(bd[0] == minval, bi[0], jnp.int32(2**30))
            for k in range(1, _CAP // 16):
                iv = jnp.minimum(iv, jnp.where(bd[k] == minval, bi[k],
                                               jnp.int32(2**30)))
            minidx = jnp.min(iv)
            lane = s % 16
            outd = jnp.where(lanes == lane, minval, outd)
            outi = jnp.where(lanes == lane, minidx, outi)
            if lane == 15:
                d2st[pl.ds(i * MAXNB + (s // 16) * 16, 16)] = outd
                colst[pl.ds(i * MAXNB + (s // 16) * 16, 16)] = outi
            for k in range(_CAP // 16):
                hit = (bd[k] == minval) & (bi[k] == minidx)
                bd[k] = jnp.where(hit, _BIGF, bd[k])
        return 0

    lax.fori_loop(0, _NPW, node_body, 0)
    pltpu.sync_copy(colst, col_hbm.at[pl.ds(base * MAXNB, _NPW * MAXNB)])
    pltpu.sync_copy(d2st, d2_hbm.at[pl.ds(base * MAXNB, _NPW * MAXNB)])


def _nbr_sc(px, py, pz):
    mesh = plsc.VectorSubcoreMesh(core_axis_name="c", subcore_axis_name="s",
                                  num_cores=2, num_subcores=16)
    f = pl.kernel(
        _nbr_body,
        out_type=[
            jax.ShapeDtypeStruct((_NPAD * MAXNB,), jnp.int32),
            jax.ShapeDtypeStruct((_NPAD * MAXNB,), jnp.float32),
        ],
        mesh=mesh,
        compiler_params=pltpu.CompilerParams(needs_layout_passes=False),
        scratch_types=[
            pltpu.VMEM((_NPAD + 16,), jnp.float32),
            pltpu.VMEM((_NPAD + 16,), jnp.float32),
            pltpu.VMEM((_NPAD + 16,), jnp.float32),
            pltpu.VMEM((_CAP,), jnp.float32),
            pltpu.VMEM((_CAP,), jnp.int32),
            pltpu.VMEM((_NPW * MAXNB,), jnp.int32),
            pltpu.VMEM((_NPW * MAXNB,), jnp.float32),
        ],
    )
    return f(px, py, pz)


def _filt_body(dist_ref, cw_ref, w0_ref, b0_ref, w1_ref, b1_ref, out_ref):
    d = dist_ref[...]
    cw = cw_ref[...]
    off = _GSTEP * lax.broadcasted_iota(jnp.int32, (1, NG), 1).astype(jnp.float32)
    attr = jnp.exp(-_GAMMA * (d - off) ** 2)
    for i in range(NINT):
        x = jnp.dot(attr, w0_ref[i], preferred_element_type=jnp.float32)
        x = _ssp(x + b0_ref[i:i + 1, :])
        x = jnp.dot(x, w1_ref[i], preferred_element_type=jnp.float32)
        x = x + b1_ref[i:i + 1, :]
        out_ref[i] = x * cw


def _filters(dist, cw, w0, b0, w1, b1):
    nb = E // BE
    return pl.pallas_call(
        _filt_body,
        grid=(nb,),
        in_specs=[
            pl.BlockSpec((BE, 1), lambda b: (b, 0)),
            pl.BlockSpec((BE, 1), lambda b: (b, 0)),
            pl.BlockSpec((NINT, NG, NFILT), lambda b: (0, 0, 0)),
            pl.BlockSpec((NINT, NFILT), lambda b: (0, 0)),
            pl.BlockSpec((NINT, NFILT, NFILT), lambda b: (0, 0, 0)),
            pl.BlockSpec((NINT, NFILT), lambda b: (0, 0)),
        ],
        out_specs=pl.BlockSpec((NINT, BE, NFILT), lambda b: (0, b, 0)),
        out_shape=jax.ShapeDtypeStruct((NINT, E, NFILT), jnp.float32),
    )(dist, cw, w0, b0, w1, b1)


def _gather_sc(hf, colp, fc_all, layer):
    def _gath_body(hf_hbm, colp_hbm, fc_hbm, agg_hbm,
                   colv, rows0, fcb0, rows1, fcb1, aggst, sem0, sem1):
        wid = lax.axis_index("s") * 2 + lax.axis_index("c")
        base = wid * _NPW
        pltpu.sync_copy(colp_hbm.at[pl.ds(base * MAXNB, _NPW * MAXNB)], colv)
        bufs = ((rows0, fcb0, sem0), (rows1, fcb1, sem1))

        def issue(i, b):
            rows_b, fcb_b, sem_b = bufs[b]
            ic = jnp.minimum(i, _NPW - 1)
            nc = jnp.minimum(base + ic, N - 1)
            pltpu.async_copy(
                hf_hbm.at[colv.at[pl.ds(ic * MAXNB, MAXNB)]], rows_b, sem_b)
            pltpu.async_copy(
                fc_hbm.at[layer, pl.ds(nc * MAXNB, MAXNB), :], fcb_b, sem_b)

        def waitpair(b):
            rows_b, fcb_b, sem_b = bufs[b]
            pltpu.make_async_copy(
                hf_hbm.at[pl.ds(0, MAXNB), :], rows_b, sem_b).wait()
            pltpu.make_async_copy(
                fc_hbm.at[layer, pl.ds(0, MAXNB), :], fcb_b, sem_b).wait()

        def compute(i, b):
            rows_b, fcb_b, _ = bufs[b]
            for c in range(NFILT // 16):
                acc = rows_b[0, pl.ds(c * 16, 16)] * fcb_b[0, pl.ds(c * 16, 16)]
                for r in range(1, MAXNB):
                    acc = acc + (rows_b[r, pl.ds(c * 16, 16)]
                                 * fcb_b[r, pl.ds(c * 16, 16)])
                aggst[pl.ds(i * NFILT + c * 16, 16)] = acc

        issue(0, 0)

        def pair_body(p, _):
            i0 = p * 2
            issue(i0 + 1, 1)
            waitpair(0)
            compute(i0, 0)
            issue(i0 + 2, 0)
            waitpair(1)
            compute(i0 + 1, 1)
            return 0

        lax.fori_loop(0, _NPW // 2, pair_body, 0)
        waitpair(0)
        pltpu.sync_copy(aggst, agg_hbm.at[pl.ds(base * NFILT, _NPW * NFILT)])

    mesh = plsc.VectorSubcoreMesh(core_axis_name="c", subcore_axis_name="s",
                                  num_cores=2, num_subcores=16)
    f = pl.kernel(
        _gath_body,
        out_type=jax.ShapeDtypeStruct((_NPAD * NFILT,), jnp.float32),
        mesh=mesh,
        compiler_params=pltpu.CompilerParams(needs_layout_passes=False),
        scratch_types=[
            pltpu.VMEM((_NPW * MAXNB,), jnp.int32),
            pltpu.VMEM((MAXNB, NFILT), jnp.float32),
            pltpu.VMEM((MAXNB, NFILT), jnp.float32),
            pltpu.VMEM((MAXNB, NFILT), jnp.float32),
            pltpu.VMEM((MAXNB, NFILT), jnp.float32),
            pltpu.VMEM((_NPW * NFILT,), jnp.float32),
            pltpu.SemaphoreType.DMA,
            pltpu.SemaphoreType.DMA,
        ],
    )
    return f(hf, colp, fc_all).reshape(_NPAD, NFILT)[:N]


def _hf_body(h_ref, w_ref, out_ref):
    out_ref[...] = jnp.dot(h_ref[...], w_ref[0],
                           preferred_element_type=jnp.float32)


def _hf(h, conv_w1, i):
    return pl.pallas_call(
        _hf_body,
        grid=(N // BH,),
        in_specs=[
            pl.BlockSpec((BH, HIDDEN), lambda b: (b, 0)),
            pl.BlockSpec((1, HIDDEN, NFILT), lambda b, _i=i: (_i, 0, 0)),
        ],
        out_specs=pl.BlockSpec((BH, NFILT), lambda b: (b, 0)),
        out_shape=jax.ShapeDtypeStruct((N, NFILT), jnp.float32),
    )(h, conv_w1)


def _upd_body(agg_ref, h_ref, w2_ref, b2_ref, lw_ref, lb_ref, out_ref):
    hc = jnp.dot(agg_ref[...], w2_ref[0], preferred_element_type=jnp.float32)
    hc = _ssp(hc + b2_ref[0:1, 0, :])
    hc = jnp.dot(hc, lw_ref[0], preferred_element_type=jnp.float32)
    out_ref[...] = h_ref[...] + hc + lb_ref[0:1, 0, :]


def _upd(agg, h, conv_w2, conv_b2, lin_w, lin_b, i):
    return pl.pallas_call(
        _upd_body,
        grid=(N // BH,),
        in_specs=[
            pl.BlockSpec((BH, NFILT), lambda b: (b, 0)),
            pl.BlockSpec((BH, HIDDEN), lambda b: (b, 0)),
            pl.BlockSpec((1, NFILT, HIDDEN), lambda b, _i=i: (_i, 0, 0)),
            pl.BlockSpec((1, 1, HIDDEN), lambda b, _i=i: (_i, 0, 0)),
            pl.BlockSpec((1, HIDDEN, HIDDEN), lambda b, _i=i: (_i, 0, 0)),
            pl.BlockSpec((1, 1, HIDDEN), lambda b, _i=i: (_i, 0, 0)),
        ],
        out_specs=pl.BlockSpec((BH, HIDDEN), lambda b: (b, 0)),
        out_shape=jax.ShapeDtypeStruct((N, HIDDEN), jnp.float32),
    )(agg, h, conv_w2, conv_b2, lin_w, lin_b)


def _readout_body(h_ref, w1_ref, b1_ref, w2_ref, b2_ref, out_ref):
    @pl.when(pl.program_id(0) == 0)
    def _():
        out_ref[...] = jnp.zeros_like(out_ref)

    x = _ssp(jnp.dot(h_ref[...], w1_ref[...],
                     preferred_element_type=jnp.float32) + b1_ref[...])
    y = jnp.dot(x, w2_ref[...], preferred_element_type=jnp.float32)
    out_ref[...] += (jnp.sum(y, axis=0, keepdims=True)
                     + BR * b2_ref[...])


def _readout(h, out_w1, out_b1, out_w2, out_b2):
    return pl.pallas_call(
        _readout_body,
        grid=(N // BR,),
        in_specs=[
            pl.BlockSpec((BR, HIDDEN), lambda b: (b, 0)),
            pl.BlockSpec((HIDDEN, HIDDEN // 2), lambda b: (0, 0)),
            pl.BlockSpec((1, HIDDEN // 2), lambda b: (0, 0)),
            pl.BlockSpec((HIDDEN // 2, 1), lambda b: (0, 0)),
            pl.BlockSpec((1, 1), lambda b: (0, 0)),
        ],
        out_specs=pl.BlockSpec((1, 1), lambda b: (0, 0)),
        out_shape=jax.ShapeDtypeStruct((1, 1), jnp.float32),
    )(h, out_w1, out_b1.reshape(1, -1), out_w2, out_b2.reshape(1, 1))


def kernel(z, pos, emb, mlp_w0, mlp_b0, mlp_w1, mlp_b1, conv_w1, conv_w2,
           conv_b2, lin_w, lin_b, out_w1, out_b1, out_w2, out_b2):
    order = jnp.argsort(pos[:, 0]).astype(jnp.int32)
    pos_s = pos[order]
    posp = jnp.concatenate(
        [pos_s, jnp.full((_NPAD - N, 3), 1e9, jnp.float32)], axis=0)
    colp, d2p = _nbr_sc(posp[:, 0], posp[:, 1], posp[:, 2])
    d2s = d2p[:E]

    valid = d2s <= CUTOFF**2
    dist = jnp.sqrt(d2s + 1e-12)
    c = 0.5 * (jnp.cos(dist * (jnp.pi / CUTOFF)) + 1.0)
    cw = c * valid.astype(jnp.float32)

    fc_all = _filters(dist.reshape(E, 1), cw.reshape(E, 1),
                      mlp_w0, mlp_b0, mlp_w1, mlp_b1)

    h = emb[z[order]]
    cb2 = conv_b2.reshape(NINT, 1, HIDDEN)
    lb = lin_b.reshape(NINT, 1, HIDDEN)
    for i in range(NINT):
        hf = _hf(h, conv_w1, i)
        agg = _gather_sc(hf, colp, fc_all, i)
        h = _upd(agg, h, conv_w2, cb2, lin_w, lb, i)

    return _readout(h, out_w1, out_b1, out_w2, out_b2)


# speedup vs baseline: 2.3509x; 1.0763x over previous
import functools

import jax
import jax.numpy as jnp
from jax import lax
from jax.experimental import pallas as pl
from jax.experimental.pallas import tpu as pltpu
from jax.experimental.pallas import tpu_sc as plsc

N = 10000
HIDDEN = 128
NFILT = 128
NINT = 6
NG = 50
CUTOFF = 5.0
MAXNB = 32
E = N * MAXNB

_LN2 = 0.6931471805599453
_GSTEP = CUTOFF / (NG - 1)
_GAMMA = 0.5 / _GSTEP**2

BE = 2000
BN = 200
BH = 2000
BR = 2000


def _ssp(x):
    return jnp.maximum(x, 0.0) + jnp.log1p(jnp.exp(-jnp.abs(x))) - _LN2


_NW = 32
_NPW = 320
_NPAD = _NW * _NPW
_NCH = 625
_CAP = 128
_BIGF = 1e30


def _nbr_body(px_hbm, py_hbm, pz_hbm, col_hbm, d2_hbm,
              xs, ys, zs, bufd, bufi, colst, d2st):
    wid = lax.axis_index("s") * 2 + lax.axis_index("c")
    base = wid * _NPW
    pltpu.sync_copy(px_hbm, xs.at[pl.ds(0, _NPAD)])
    pltpu.sync_copy(py_hbm, ys.at[pl.ds(0, _NPAD)])
    pltpu.sync_copy(pz_hbm, zs.at[pl.ds(0, _NPAD)])
    lanes = lax.broadcasted_iota(jnp.int32, (16,), 0)

    def _bsearch(target, right):
        def it(_, lh):
            lo, hi = lh
            mid = (lo + hi) // 2
            v = xs[pl.ds(mid, 16)][0]
            go = (v <= target) if right else (v < target)
            return (jnp.where(go, mid + 1, lo), jnp.where(go, hi, mid))

        lo, _hi = lax.fori_loop(0, 14, it, (jnp.int32(0), jnp.int32(N)))
        return lo

    def node_body(i, _):
        n = base + i
        x0 = xs[pl.ds(n, 16)][0]
        y0 = ys[pl.ds(n, 16)][0]
        z0 = zs[pl.ds(n, 16)][0]
        jc0 = _bsearch(x0 - CUTOFF, False) // 16
        jc1 = (_bsearch(x0 + CUTOFF, True) + 15) // 16
        for k in range(_CAP // 16):
            bufd[pl.ds(k * 16, 16)] = jnp.full((16,), _BIGF, jnp.float32)
            bufi[pl.ds(k * 16, 16)] = jnp.full((16,), i, jnp.int32)

        def cand_body(j, cnt):
            c0 = j * 16
            dx = xs[pl.ds(c0, 16)] - x0
            dy = ys[pl.ds(c0, 16)] - y0
            dz = zs[pl.ds(c0, 16)] - z0
            d2 = dx * dx + dy * dy + dz * dz
            ids = c0 + lanes
            m = (d2 <= CUTOFF**2) & (ids != n)
            pos = cnt + plsc.cumsum(m.astype(jnp.int32)) - m.astype(jnp.int32)
            m = m & (pos < _CAP)
            plsc.store_scatter(bufd, [pos], d2, mask=m)
            plsc.store_scatter(bufi, [pos], ids, mask=m)
            return cnt + plsc.all_reduce_population_count(m)

        lax.fori_loop(jc0, jc1, cand_body, jnp.zeros((16,), jnp.int32))

        bd = [bufd[pl.ds(k * 16, 16)] for k in range(_CAP // 16)]
        bi = [bufi[pl.ds(k * 16, 16)] for k in range(_CAP // 16)]
        outd = jnp.zeros((16,), jnp.float32)
        outi = jnp.zeros((16,), jnp.int32)
        for s in range(MAXNB):
            mv = bd[0]
            for k in range(1, _CAP // 16):
                mv = jnp.minimum(mv, bd[k])
            minval = jnp.min(mv)
            iv = jnp.where(bd[0] == minval, bi[0], jnp.int32(2**30))
            for k in range(1, _CAP // 16):
                iv = jnp.minimum(iv, jnp.where(bd[k] == minval, bi[k],
                                               jnp.int32(2**30)))
            minidx = jnp.min(iv)
            lane = s % 16
            outd = jnp.where(lanes == lane, minval, outd)
            outi = jnp.where(lanes == lane, minidx, outi)
            if lane == 15:
                d2st[pl.ds(i * MAXNB + (s // 16) * 16, 16)] = outd
                colst[pl.ds(i * MAXNB + (s // 16) * 16, 16)] = outi
            for k in range(_CAP // 16):
                hit = (bd[k] == minval) & (bi[k] == minidx)
                bd[k] = jnp.where(hit, _BIGF, bd[k])
        return 0

    lax.fori_loop(0, _NPW, node_body, 0)
    pltpu.sync_copy(colst, col_hbm.at[pl.ds(base * MAXNB, _NPW * MAXNB)])
    pltpu.sync_copy(d2st, d2_hbm.at[pl.ds(base * MAXNB, _NPW * MAXNB)])


def _nbr_sc(px, py, pz):
    mesh = plsc.VectorSubcoreMesh(core_axis_name="c", subcore_axis_name="s",
                                  num_cores=2, num_subcores=16)
    f = pl.kernel(
        _nbr_body,
        out_type=[
            jax.ShapeDtypeStruct((_NPAD * MAXNB,), jnp.int32),
            jax.ShapeDtypeStruct((_NPAD * MAXNB,), jnp.float32),
        ],
        mesh=mesh,
        compiler_params=pltpu.CompilerParams(needs_layout_passes=False),
        scratch_types=[
            pltpu.VMEM((_NPAD + 16,), jnp.float32),
            pltpu.VMEM((_NPAD + 16,), jnp.float32),
            pltpu.VMEM((_NPAD + 16,), jnp.float32),
            pltpu.VMEM((_CAP,), jnp.float32),
            pltpu.VMEM((_CAP,), jnp.int32),
            pltpu.VMEM((_NPW * MAXNB,), jnp.int32),
            pltpu.VMEM((_NPW * MAXNB,), jnp.float32),
        ],
    )
    return f(px, py, pz)


def _filt_body(dist_ref, cw_ref, w0_ref, b0_ref, w1_ref, b1_ref, out_ref):
    d = dist_ref[...]
    cw = cw_ref[...]
    off = _GSTEP * lax.broadcasted_iota(jnp.int32, (1, NG), 1).astype(jnp.float32)
    attr = jnp.exp(-_GAMMA * (d - off) ** 2)
    x = jnp.dot(attr, w0_ref[0], preferred_element_type=jnp.float32)
    x = _ssp(x + b0_ref[0:1, 0, :])
    x = jnp.dot(x, w1_ref[0], preferred_element_type=jnp.float32)
    x = x + b1_ref[0:1, 0, :]
    out_ref[...] = x * cw


def _filters(dist, cw, w0, b0, w1, b1, i):
    nb = E // BE
    return pl.pallas_call(
        _filt_body,
        grid=(nb,),
        in_specs=[
            pl.BlockSpec((BE, 1), lambda b: (b, 0)),
            pl.BlockSpec((BE, 1), lambda b: (b, 0)),
            pl.BlockSpec((1, NG, NFILT), lambda b, _i=i: (_i, 0, 0)),
            pl.BlockSpec((1, 1, NFILT), lambda b, _i=i: (_i, 0, 0)),
            pl.BlockSpec((1, NFILT, NFILT), lambda b, _i=i: (_i, 0, 0)),
            pl.BlockSpec((1, 1, NFILT), lambda b, _i=i: (_i, 0, 0)),
        ],
        out_specs=pl.BlockSpec((BE, NFILT), lambda b: (b, 0)),
        out_shape=jax.ShapeDtypeStruct((E, NFILT), jnp.float32),
    )(dist, cw, w0, b0.reshape(NINT, 1, NFILT), w1, b1.reshape(NINT, 1, NFILT))


_NBUF = 4


def _gather_sc(hf, colp, fc):
    def _gath_body(hf_hbm, colp_hbm, fc_hbm, agg_hbm, colv,
                   rows0, fcb0, rows1, fcb1, rows2, fcb2, rows3, fcb3,
                   aggst, sem0, sem1, sem2, sem3):
        wid = lax.axis_index("s") * 2 + lax.axis_index("c")
        base = wid * _NPW
        pltpu.sync_copy(colp_hbm.at[pl.ds(base * MAXNB, _NPW * MAXNB)], colv)
        bufs = ((rows0, fcb0, sem0), (rows1, fcb1, sem1),
                (rows2, fcb2, sem2), (rows3, fcb3, sem3))

        def issue(i, b):
            rows_b, fcb_b, sem_b = bufs[b]
            ic = jnp.minimum(i, _NPW - 1)
            nc = jnp.minimum(base + ic, N - 1)
            pltpu.async_copy(
                hf_hbm.at[colv.at[pl.ds(ic * MAXNB, MAXNB)]], rows_b, sem_b)
            pltpu.async_copy(
                fc_hbm.at[pl.ds(nc * MAXNB, MAXNB), :], fcb_b, sem_b)

        def waitpair(b):
            rows_b, fcb_b, sem_b = bufs[b]
            pltpu.make_async_copy(
                hf_hbm.at[pl.ds(0, MAXNB), :], rows_b, sem_b).wait()
            pltpu.make_async_copy(
                fc_hbm.at[pl.ds(0, MAXNB), :], fcb_b, sem_b).wait()

        def compute(i, b):
            rows_b, fcb_b, _ = bufs[b]
            for c in range(NFILT // 16):
                acc = rows_b[0, pl.ds(c * 16, 16)] * fcb_b[0, pl.ds(c * 16, 16)]
                for r in range(1, MAXNB):
                    acc = acc + (rows_b[r, pl.ds(c * 16, 16)]
                                 * fcb_b[r, pl.ds(c * 16, 16)])
                aggst[pl.ds(i * NFILT + c * 16, 16)] = acc

        for b in range(_NBUF):
            issue(b, b)

        def ring_body(p, _):
            i0 = p * _NBUF
            for b in range(_NBUF):
                waitpair(b)
                compute(i0 + b, b)
                issue(i0 + b + _NBUF, b)
            return 0

        lax.fori_loop(0, _NPW // _NBUF - 1, ring_body, 0)
        i0 = _NPW - _NBUF
        for b in range(_NBUF):
            waitpair(b)
            compute(i0 + b, b)
        pltpu.sync_copy(aggst, agg_hbm.at[pl.ds(base * NFILT, _NPW * NFILT)])

    mesh = plsc.VectorSubcoreMesh(core_axis_name="c", subcore_axis_name="s",
                                  num_cores=2, num_subcores=16)
    f = pl.kernel(
        _gath_body,
        out_type=jax.ShapeDtypeStruct((_NPAD * NFILT,), jnp.float32),
        mesh=mesh,
        compiler_params=pltpu.CompilerParams(needs_layout_passes=False),
        scratch_types=(
            [pltpu.VMEM((_NPW * MAXNB,), jnp.int32)]
            + [pltpu.VMEM((MAXNB, NFILT), jnp.float32)] * (2 * _NBUF)
            + [pltpu.VMEM((_NPW * NFILT,), jnp.float32)]
            + [pltpu.SemaphoreType.DMA] * _NBUF
        ),
    )
    return f(hf, colp, fc).reshape(_NPAD, NFILT)[:N]


def _hf_body(h_ref, w_ref, out_ref):
    out_ref[...] = jnp.dot(h_ref[...], w_ref[0],
                           preferred_element_type=jnp.float32)


def _hf(h, conv_w1, i):
    return pl.pallas_call(
        _hf_body,
        grid=(N // BH,),
        in_specs=[
            pl.BlockSpec((BH, HIDDEN), lambda b: (b, 0)),
            pl.BlockSpec((1, HIDDEN, NFILT), lambda b, _i=i: (_i, 0, 0)),
        ],
        out_specs=pl.BlockSpec((BH, NFILT), lambda b: (b, 0)),
        out_shape=jax.ShapeDtypeStruct((N, NFILT), jnp.float32),
    )(h, conv_w1)


def _upd_body(agg_ref, h_ref, w2_ref, b2_ref, lw_ref, lb_ref, out_ref):
    hc = jnp.dot(agg_ref[...], w2_ref[0], preferred_element_type=jnp.float32)
    hc = _ssp(hc + b2_ref[0:1, 0, :])
    hc = jnp.dot(hc, lw_ref[0], preferred_element_type=jnp.float32)
    out_ref[...] = h_ref[...] + hc + lb_ref[0:1, 0, :]


def _upd(agg, h, conv_w2, conv_b2, lin_w, lin_b, i):
    return pl.pallas_call(
        _upd_body,
        grid=(N // BH,),
        in_specs=[
            pl.BlockSpec((BH, NFILT), lambda b: (b, 0)),
            pl.BlockSpec((BH, HIDDEN), lambda b: (b, 0)),
            pl.BlockSpec((1, NFILT, HIDDEN), lambda b, _i=i: (_i, 0, 0)),
            pl.BlockSpec((1, 1, HIDDEN), lambda b, _i=i: (_i, 0, 0)),
            pl.BlockSpec((1, HIDDEN, HIDDEN), lambda b, _i=i: (_i, 0, 0)),
            pl.BlockSpec((1, 1, HIDDEN), lambda b, _i=i: (_i, 0, 0)),
        ],
        out_specs=pl.BlockSpec((BH, HIDDEN), lambda b: (b, 0)),
        out_shape=jax.ShapeDtypeStruct((N, HIDDEN), jnp.float32),
    )(agg, h, conv_w2, conv_b2, lin_w, lin_b)


def _readout_body(h_ref, w1_ref, b1_ref, w2_ref, b2_ref, out_ref):
    @pl.when(pl.program_id(0) == 0)
    def _():
        out_ref[...] = jnp.zeros_like(out_ref)

    x = _ssp(jnp.dot(h_ref[...], w1_ref[...],
                     preferred_element_type=jnp.float32) + b1_ref[...])
    y = jnp.dot(x, w2_ref[...], preferred_element_type=jnp.float32)
    out_ref[...] += (jnp.sum(y, axis=0, keepdims=True)
                     + BR * b2_ref[...])


def _readout(h, out_w1, out_b1, out_w2, out_b2):
    return pl.pallas_call(
        _readout_body,
        grid=(N // BR,),
        in_specs=[
            pl.BlockSpec((BR, HIDDEN), lambda b: (b, 0)),
            pl.BlockSpec((HIDDEN, HIDDEN // 2), lambda b: (0, 0)),
            pl.BlockSpec((1, HIDDEN // 2), lambda b: (0, 0)),
            pl.BlockSpec((HIDDEN // 2, 1), lambda b: (0, 0)),
            pl.BlockSpec((1, 1), lambda b: (0, 0)),
        ],
        out_specs=pl.BlockSpec((1, 1), lambda b: (0, 0)),
        out_shape=jax.ShapeDtypeStruct((1, 1), jnp.float32),
    )(h, out_w1, out_b1.reshape(1, -1), out_w2, out_b2.reshape(1, 1))


def kernel(z, pos, emb, mlp_w0, mlp_b0, mlp_w1, mlp_b1, conv_w1, conv_w2,
           conv_b2, lin_w, lin_b, out_w1, out_b1, out_w2, out_b2):
    order = jnp.argsort(pos[:, 0]).astype(jnp.int32)
    pos_s = pos[order]
    posp = jnp.concatenate(
        [pos_s, jnp.full((_NPAD - N, 3), 1e9, jnp.float32)], axis=0)
    colp, d2p = _nbr_sc(posp[:, 0], posp[:, 1], posp[:, 2])
    d2s = d2p[:E]

    valid = d2s <= CUTOFF**2
    dist = jnp.sqrt(d2s + 1e-12)
    c = 0.5 * (jnp.cos(dist * (jnp.pi / CUTOFF)) + 1.0)
    cw = c * valid.astype(jnp.float32)

    dist2 = dist.reshape(E, 1)
    cw2 = cw.reshape(E, 1)
    fc_next = _filters(dist2, cw2, mlp_w0, mlp_b0, mlp_w1, mlp_b1, 0)

    h = emb[z[order]]
    cb2 = conv_b2.reshape(NINT, 1, HIDDEN)
    lb = lin_b.reshape(NINT, 1, HIDDEN)
    for i in range(NINT):
        hf = _hf(h, conv_w1, i)
        fc_i = fc_next
        agg = _gather_sc(hf, colp, fc_i)
        if i + 1 < NINT:
            fc_next = _filters(dist2, cw2, mlp_w0, mlp_b0, mlp_w1, mlp_b1,
                               i + 1)
        h = _upd(agg, h, conv_w2, cb2, lin_w, lb, i)

    return _readout(h, out_w1, out_b1, out_w2, out_b2)


# speedup vs baseline: 2.3808x; 1.0127x over previous
import functools

import jax
import jax.numpy as jnp
from jax import lax
from jax.experimental import pallas as pl
from jax.experimental.pallas import tpu as pltpu
from jax.experimental.pallas import tpu_sc as plsc

N = 10000
HIDDEN = 128
NFILT = 128
NINT = 6
NG = 50
CUTOFF = 5.0
MAXNB = 32
E = N * MAXNB

_LN2 = 0.6931471805599453
_GSTEP = CUTOFF / (NG - 1)
_GAMMA = 0.5 / _GSTEP**2

BE = 2000
BN = 200
BH = 2000
BR = 2000


def _ssp(x):
    return jnp.maximum(x, 0.0) + jnp.log1p(jnp.exp(-jnp.abs(x))) - _LN2


_NW = 32
_NPW = 320
_NPAD = _NW * _NPW
_NCH = 625
_CAP = 128
_BIGF = 1e30


def _nbr_body(px_hbm, py_hbm, pz_hbm, col_hbm, d2_hbm,
              xs, ys, zs, bufd, bufi, colst, d2st):
    wid = lax.axis_index("s") * 2 + lax.axis_index("c")
    base = wid * _NPW
    pltpu.sync_copy(px_hbm, xs.at[pl.ds(0, _NPAD)])
    pltpu.sync_copy(py_hbm, ys.at[pl.ds(0, _NPAD)])
    pltpu.sync_copy(pz_hbm, zs.at[pl.ds(0, _NPAD)])
    lanes = lax.broadcasted_iota(jnp.int32, (16,), 0)

    def _bsearch(target, right):
        def it(_, lh):
            lo, hi = lh
            mid = (lo + hi) // 2
            v = xs[pl.ds(mid, 16)][0]
            go = (v <= target) if right else (v < target)
            return (jnp.where(go, mid + 1, lo), jnp.where(go, hi, mid))

        lo, _hi = lax.fori_loop(0, 14, it, (jnp.int32(0), jnp.int32(N)))
        return lo

    def node_body(i, _):
        n = base + i
        x0 = xs[pl.ds(n, 16)][0]
        y0 = ys[pl.ds(n, 16)][0]
        z0 = zs[pl.ds(n, 16)][0]
        jc0 = _bsearch(x0 - CUTOFF, False) // 16
        jc1 = (_bsearch(x0 + CUTOFF, True) + 15) // 16
        for k in range(_CAP // 16):
            bufd[pl.ds(k * 16, 16)] = jnp.full((16,), _BIGF, jnp.float32)
            bufi[pl.ds(k * 16, 16)] = jnp.full((16,), i, jnp.int32)

        def chunk(c0, cnt):
            dx = xs[pl.ds(c0, 16)] - x0
            dy = ys[pl.ds(c0, 16)] - y0
            dz = zs[pl.ds(c0, 16)] - z0
            d2 = dx * dx + dy * dy + dz * dz
            ids = c0 + lanes
            m = (d2 <= CUTOFF**2) & (ids != n)
            pc = plsc.all_reduce_population_count(m)
            pos = cnt + plsc.cumsum(m.astype(jnp.int32)) - m.astype(jnp.int32)
            m = m & (pos < _CAP)
            plsc.store_scatter(bufd, [pos], d2, mask=m)
            plsc.store_scatter(bufi, [pos], ids, mask=m)
            return cnt + pc

        def cand_body(p, cnt):
            c0 = (jc0 + p * 2) * 16
            cnt = chunk(c0, cnt)
            cnt = chunk(c0 + 16, cnt)
            return cnt

        lax.fori_loop(0, (jc1 - jc0 + 1) // 2, cand_body,
                      jnp.zeros((16,), jnp.int32))

        bd = [bufd[pl.ds(k * 16, 16)] for k in range(_CAP // 16)]
        bi = [bufi[pl.ds(k * 16, 16)] for k in range(_CAP // 16)]
        outd = jnp.zeros((16,), jnp.float32)
        outi = jnp.zeros((16,), jnp.int32)
        for s in range(MAXNB):
            mv = bd[0]
            for k in range(1, _CAP // 16):
                mv = jnp.minimum(mv, bd[k])
            minval = jnp.min(mv)
            iv = jnp.where(bd[0] == minval, bi[0], jnp.int32(2**30))
            for k in range(1, _CAP // 16):
                iv = jnp.minimum(iv, jnp.where(bd[k] == minval, bi[k],
                                               jnp.int32(2**30)))
            minidx = jnp.min(iv)
            lane = s % 16
            outd = jnp.where(lanes == lane, minval, outd)
            outi = jnp.where(lanes == lane, minidx, outi)
            if lane == 15:
                d2st[pl.ds(i * MAXNB + (s // 16) * 16, 16)] = outd
                colst[pl.ds(i * MAXNB + (s // 16) * 16, 16)] = outi
            for k in range(_CAP // 16):
                hit = (bd[k] == minval) & (bi[k] == minidx)
                bd[k] = jnp.where(hit, _BIGF, bd[k])
        return 0

    lax.fori_loop(0, _NPW, node_body, 0)
    pltpu.sync_copy(colst, col_hbm.at[pl.ds(base * MAXNB, _NPW * MAXNB)])
    pltpu.sync_copy(d2st, d2_hbm.at[pl.ds(base * MAXNB, _NPW * MAXNB)])


def _nbr_sc(px, py, pz):
    mesh = plsc.VectorSubcoreMesh(core_axis_name="c", subcore_axis_name="s",
                                  num_cores=2, num_subcores=16)
    f = pl.kernel(
        _nbr_body,
        out_type=[
            jax.ShapeDtypeStruct((_NPAD * MAXNB,), jnp.int32),
            jax.ShapeDtypeStruct((_NPAD * MAXNB,), jnp.float32),
        ],
        mesh=mesh,
        compiler_params=pltpu.CompilerParams(needs_layout_passes=False),
        scratch_types=[
            pltpu.VMEM((_NPAD + 16,), jnp.float32),
            pltpu.VMEM((_NPAD + 16,), jnp.float32),
            pltpu.VMEM((_NPAD + 16,), jnp.float32),
            pltpu.VMEM((_CAP,), jnp.float32),
            pltpu.VMEM((_CAP,), jnp.int32),
            pltpu.VMEM((_NPW * MAXNB,), jnp.int32),
            pltpu.VMEM((_NPW * MAXNB,), jnp.float32),
        ],
    )
    return f(px, py, pz)


def _filt_body(dist_ref, cw_ref, w0_ref, b0_ref, w1_ref, b1_ref, out_ref):
    d = dist_ref[...]
    cw = cw_ref[...]
    off = _GSTEP * lax.broadcasted_iota(jnp.int32, (1, NG), 1).astype(jnp.float32)
    attr = jnp.exp(-_GAMMA * (d - off) ** 2)
    x = jnp.dot(attr, w0_ref[0], preferred_element_type=jnp.float32)
    x = _ssp(x + b0_ref[0:1, 0, :])
    x = jnp.dot(x, w1_ref[0], preferred_element_type=jnp.float32)
    x = x + b1_ref[0:1, 0, :]
    out_ref[...] = x * cw


def _filters(dist, cw, w0, b0, w1, b1, i):
    nb = E // BE
    return pl.pallas_call(
        _filt_body,
        grid=(nb,),
        in_specs=[
            pl.BlockSpec((BE, 1), lambda b: (b, 0)),
            pl.BlockSpec((BE, 1), lambda b: (b, 0)),
            pl.BlockSpec((1, NG, NFILT), lambda b, _i=i: (_i, 0, 0)),
            pl.BlockSpec((1, 1, NFILT), lambda b, _i=i: (_i, 0, 0)),
            pl.BlockSpec((1, NFILT, NFILT), lambda b, _i=i: (_i, 0, 0)),
            pl.BlockSpec((1, 1, NFILT), lambda b, _i=i: (_i, 0, 0)),
        ],
        out_specs=pl.BlockSpec((BE, NFILT), lambda b: (b, 0)),
        out_shape=jax.ShapeDtypeStruct((E, NFILT), jnp.float32),
    )(dist, cw, w0, b0.reshape(NINT, 1, NFILT), w1, b1.reshape(NINT, 1, NFILT))


_NBUF = 4


def _gather_sc(hf, colp, fc):
    def _gath_body(hf_hbm, colp_hbm, fc_hbm, agg_hbm, colv,
                   rows0, fcb0, rows1, fcb1, rows2, fcb2, rows3, fcb3,
                   aggst, sem0, sem1, sem2, sem3):
        sid = lax.axis_index("s")
        wid = sid * 2 + lax.axis_index("c")
        base = wid * _NPW
        pltpu.sync_copy(colp_hbm.at[pl.ds(base * MAXNB, _NPW * MAXNB)], colv)
        bufs = ((rows0, fcb0, sem0), (rows1, fcb1, sem1),
                (rows2, fcb2, sem2), (rows3, fcb3, sem3))

        def issue(i, b):
            rows_b, fcb_b, sem_b = bufs[b]
            ic = jnp.minimum(i, _NPW - 1)
            nc = jnp.minimum(base + ic, N - 1)
            pltpu.async_copy(
                hf_hbm.at[colv.at[pl.ds(ic * MAXNB, MAXNB)]], rows_b, sem_b)
            pltpu.async_copy(
                fc_hbm.at[pl.ds(nc * MAXNB, MAXNB), :], fcb_b, sem_b)

        def waitpair(b):
            rows_b, fcb_b, sem_b = bufs[b]
            pltpu.make_async_copy(
                hf_hbm.at[pl.ds(0, MAXNB), :], rows_b, sem_b).wait()
            pltpu.make_async_copy(
                fc_hbm.at[pl.ds(0, MAXNB), :], fcb_b, sem_b).wait()

        def compute(i, b):
            rows_b, fcb_b, _ = bufs[b]
            for c in range(NFILT // 16):
                acc = rows_b[0, pl.ds(c * 16, 16)] * fcb_b[0, pl.ds(c * 16, 16)]
                for r in range(1, MAXNB):
                    acc = acc + (rows_b[r, pl.ds(c * 16, 16)]
                                 * fcb_b[r, pl.ds(c * 16, 16)])
                aggst[pl.ds(i * NFILT + c * 16, 16)] = acc

        for b in range(_NBUF):
            issue(b, b)

        def ring_body(p, _):
            i0 = p * _NBUF
            for b in range(_NBUF):
                waitpair(b)
                compute(i0 + b, b)
                issue(i0 + b + _NBUF, b)
            return 0

        lax.fori_loop(0, _NPW // _NBUF - 1, ring_body, 0)
        i0 = _NPW - _NBUF
        for b in range(_NBUF):
            waitpair(b)
            compute(i0 + b, b)
        pltpu.sync_copy(aggst, agg_hbm.at[pl.ds(base * NFILT, _NPW * NFILT)])

    mesh = plsc.VectorSubcoreMesh(core_axis_name="c", subcore_axis_name="s",
                                  num_cores=2, num_subcores=16)
    f = pl.kernel(
        _gath_body,
        out_type=jax.ShapeDtypeStruct((_NPAD * NFILT,), jnp.float32),
        mesh=mesh,
        compiler_params=pltpu.CompilerParams(needs_layout_passes=False),
        scratch_types=(
            [pltpu.VMEM((_NPW * MAXNB,), jnp.int32)]
            + [pltpu.VMEM((MAXNB, NFILT), jnp.float32)] * (2 * _NBUF)
            + [pltpu.VMEM((_NPW * NFILT,), jnp.float32)]
            + [pltpu.SemaphoreType.DMA] * _NBUF
        ),
    )
    return f(hf, colp, fc).reshape(_NPAD, NFILT)[:N]


def _hf_body(h_ref, w_ref, out_ref):
    out_ref[...] = jnp.dot(h_ref[...], w_ref[0],
                           preferred_element_type=jnp.float32)


def _hf(h, conv_w1, i):
    return pl.pallas_call(
        _hf_body,
        grid=(N // BH,),
        in_specs=[
            pl.BlockSpec((BH, HIDDEN), lambda b: (b, 0)),
            pl.BlockSpec((1, HIDDEN, NFILT), lambda b, _i=i: (_i, 0, 0)),
        ],
        out_specs=pl.BlockSpec((BH, NFILT), lambda b: (b, 0)),
        out_shape=jax.ShapeDtypeStruct((N, NFILT), jnp.float32),
    )(h, conv_w1)


def _upd_body(agg_ref, h_ref, w2_ref, b2_ref, lw_ref, lb_ref, out_ref):
    hc = jnp.dot(agg_ref[...], w2_ref[0], preferred_element_type=jnp.float32)
    hc = _ssp(hc + b2_ref[0:1, 0, :])
    hc = jnp.dot(hc, lw_ref[0], preferred_element_type=jnp.float32)
    out_ref[...] = h_ref[...] + hc + lb_ref[0:1, 0, :]


def _upd(agg, h, conv_w2, conv_b2, lin_w, lin_b, i):
    return pl.pallas_call(
        _upd_body,
        grid=(N // BH,),
        in_specs=[
            pl.BlockSpec((BH, NFILT), lambda b: (b, 0)),
            pl.BlockSpec((BH, HIDDEN), lambda b: (b, 0)),
            pl.BlockSpec((1, NFILT, HIDDEN), lambda b, _i=i: (_i, 0, 0)),
            pl.BlockSpec((1, 1, HIDDEN), lambda b, _i=i: (_i, 0, 0)),
            pl.BlockSpec((1, HIDDEN, HIDDEN), lambda b, _i=i: (_i, 0, 0)),
            pl.BlockSpec((1, 1, HIDDEN), lambda b, _i=i: (_i, 0, 0)),
        ],
        out_specs=pl.BlockSpec((BH, HIDDEN), lambda b: (b, 0)),
        out_shape=jax.ShapeDtypeStruct((N, HIDDEN), jnp.float32),
    )(agg, h, conv_w2, conv_b2, lin_w, lin_b)


def _readout_body(h_ref, w1_ref, b1_ref, w2_ref, b2_ref, out_ref):
    @pl.when(pl.program_id(0) == 0)
    def _():
        out_ref[...] = jnp.zeros_like(out_ref)

    x = _ssp(jnp.dot(h_ref[...], w1_ref[...],
                     preferred_element_type=jnp.float32) + b1_ref[...])
    y = jnp.dot(x, w2_ref[...], preferred_element_type=jnp.float32)
    out_ref[...] += (jnp.sum(y, axis=0, keepdims=True)
                     + BR * b2_ref[...])


def _readout(h, out_w1, out_b1, out_w2, out_b2):
    return pl.pallas_call(
        _readout_body,
        grid=(N // BR,),
        in_specs=[
            pl.BlockSpec((BR, HIDDEN), lambda b: (b, 0)),
            pl.BlockSpec((HIDDEN, HIDDEN // 2), lambda b: (0, 0)),
            pl.BlockSpec((1, HIDDEN // 2), lambda b: (0, 0)),
            pl.BlockSpec((HIDDEN // 2, 1), lambda b: (0, 0)),
            pl.BlockSpec((1, 1), lambda b: (0, 0)),
        ],
        out_specs=pl.BlockSpec((1, 1), lambda b: (0, 0)),
        out_shape=jax.ShapeDtypeStruct((1, 1), jnp.float32),
    )(h, out_w1, out_b1.reshape(1, -1), out_w2, out_b2.reshape(1, 1))


def kernel(z, pos, emb, mlp_w0, mlp_b0, mlp_w1, mlp_b1, conv_w1, conv_w2,
           conv_b2, lin_w, lin_b, out_w1, out_b1, out_w2, out_b2):
    order = jnp.argsort(pos[:, 0]).astype(jnp.int32)
    pos_s = pos[order]
    posp = jnp.concatenate(
        [pos_s, jnp.full((_NPAD - N, 3), 1e9, jnp.float32)], axis=0)
    colp, d2p = _nbr_sc(posp[:, 0], posp[:, 1], posp[:, 2])
    d2s = d2p[:E]

    valid = d2s <= CUTOFF**2
    dist = jnp.sqrt(d2s + 1e-12)
    c = 0.5 * (jnp.cos(dist * (jnp.pi / CUTOFF)) + 1.0)
    cw = c * valid.astype(jnp.float32)

    dist2 = dist.reshape(E, 1)
    cw2 = cw.reshape(E, 1)
    fc_next = _filters(dist2, cw2, mlp_w0, mlp_b0, mlp_w1, mlp_b1, 0)

    h = emb[z[order]]
    cb2 = conv_b2.reshape(NINT, 1, HIDDEN)
    lb = lin_b.reshape(NINT, 1, HIDDEN)
    for i in range(NINT):
        hf = _hf(h, conv_w1, i)
        fc_i = fc_next
        agg = _gather_sc(hf, colp, fc_i)
        if i + 1 < NINT:
            fc_next = _filters(dist2, cw2, mlp_w0, mlp_b0, mlp_w1, mlp_b1,
                               i + 1)
        h = _upd(agg, h, conv_w2, cb2, lin_w, lb, i)

    return _readout(h, out_w1, out_b1, out_w2, out_b2)
